# Initial kernel scaffold; baseline (speedup 1.0000x reference)
import jax
import jax.numpy as jnp
from jax.experimental import pallas as pl


def kernel(edge_index, h_train, h_ori, Wq, bq, Wk, bk, Wv, bv):
    raise NotImplementedError("write your pallas kernel here")


# speedup vs baseline: 6.9399x; 6.9399x over previous
import math

import jax
import jax.numpy as jnp
from jax import lax
from jax.experimental import pallas as pl
from jax.experimental.pallas import tpu as pltpu
from jax.experimental.pallas import tpu_sc as plsc

N_NODES = 10000
N_PAD = 10240
D = 128
E = 320000
K_HOPS = 8
NC = 2
NS = 16
NW = NC * NS
E_W = E // NW
C = 80
NCH = E_W // C
N_R = 10240
R_T = N_R // NS
D_T = N_PAD // NS

_SCALE = 1.0 / math.sqrt(D)
_HI = lax.Precision.HIGHEST

B = 2000
GRID = N_NODES // B


def _sc_mesh():
    return plsc.VectorSubcoreMesh(core_axis_name="c", subcore_axis_name="s")


def _deg_body(row_hbm, out_hbm, row_v, ones_v, zb, acc1):
    cid = lax.axis_index("c")
    sid = lax.axis_index("s")
    wid = cid * NS + sid

    for i in range(C // 16):
        ones_v[pl.ds(i * 16, 16)] = jnp.ones((16,), jnp.float32)
    for i in range(D_T // 16):
        zb[pl.ds(i * 16, 16)] = jnp.zeros((16,), jnp.float32)
    pltpu.sync_copy(zb, acc1.at[pl.ds(sid * D_T, D_T)])
    pltpu.sync_copy(row_hbm.at[wid], row_v)
    plsc.subcore_barrier()

    def chunk(j, carry):
        pltpu.sync_copy(ones_v, acc1.at[row_v.at[j]], add=True)
        return carry

    lax.fori_loop(0, NCH, chunk, 0)
    plsc.subcore_barrier()
    pltpu.sync_copy(acc1.at[pl.ds(sid * D_T, D_T)],
                    out_hbm.at[cid, pl.ds(sid * D_T, D_T)])


def _sc_deg(row):
    return pl.kernel(
        _deg_body,
        mesh=_sc_mesh(),
        out_type=jax.ShapeDtypeStruct((NC, N_PAD), jnp.float32),
        scratch_types=[
            pltpu.VMEM((NCH, C), jnp.int32),
            pltpu.VMEM((C,), jnp.float32),
            pltpu.VMEM((D_T,), jnp.float32),
            pltpu.VMEM_SHARED((N_PAD,), jnp.float32),
        ],
    )(row)


def _hop_body(y_hbm, row_hbm, col_hbm, out_hbm, row_v, col_v, gbuf, acc, sem):
    cid = lax.axis_index("c")
    sid = lax.axis_index("s")
    wid = cid * NS + sid

    def zi(i, carry):
        r = i // 8
        c0 = (i - r * 8) * 16
        gbuf[r, pl.ds(c0, 16)] = jnp.zeros((16,), jnp.float32)
        return carry

    lax.fori_loop(0, C * 8, zi, 0)

    def zc(k, carry):
        pltpu.sync_copy(gbuf, acc.at[pl.ds(sid * R_T + k * C, C)])
        return carry

    lax.fori_loop(0, R_T // C, zc, 0)

    pltpu.sync_copy(row_hbm.at[wid], row_v)
    pltpu.sync_copy(col_hbm.at[wid], col_v)
    plsc.subcore_barrier()

    def chunk(j, carry):
        pltpu.async_copy(y_hbm.at[row_v.at[j]], gbuf, sem).wait()
        pltpu.sync_copy(gbuf, acc.at[col_v.at[j]], add=True)
        return carry

    lax.fori_loop(0, NCH, chunk, 0)
    plsc.subcore_barrier()
    pltpu.sync_copy(acc.at[pl.ds(sid * R_T, R_T)],
                    out_hbm.at[cid, pl.ds(sid * R_T, R_T)])


def _sc_hop(y, row, col):
    return pl.kernel(
        _hop_body,
        mesh=_sc_mesh(),
        out_type=jax.ShapeDtypeStruct((NC, N_R, D), jnp.float32),
        scratch_types=[
            pltpu.VMEM((NCH, C), jnp.int32),
            pltpu.VMEM((NCH, C), jnp.int32),
            pltpu.VMEM((C, D), jnp.float32),
            pltpu.VMEM_SHARED((N_R, D), jnp.float32),
            pltpu.SemaphoreType.DMA,
        ],
    )(y, row, col)


def _prep_body(dp0, dp1, ht, ho, dinv_o, y0_o, hsum_o):
    i = pl.program_id(0)
    deg = jnp.maximum(dp0[...] + dp1[...], 1.0)
    dinv = lax.rsqrt(deg)
    dinv_o[...] = dinv
    y0_o[...] = ht[...] * dinv

    @pl.when(i == 0)
    def _():
        hsum_o[...] = jnp.zeros_like(hsum_o)

    hsum_o[...] += jnp.sum(ho[...], axis=0, keepdims=True)


def _tc_prep(dp0, dp1, h_train, h_ori):
    col1 = pl.BlockSpec((B, 1), lambda i: (i, 0))
    full = pl.BlockSpec((B, D), lambda i: (i, 0))
    return pl.pallas_call(
        _prep_body,
        grid=(GRID,),
        in_specs=[col1, col1, full, full],
        out_specs=[col1, full, pl.BlockSpec((1, D), lambda i: (0, 0))],
        out_shape=[
            jax.ShapeDtypeStruct((N_NODES, 1), jnp.float32),
            jax.ShapeDtypeStruct((N_NODES, D), jnp.float32),
            jax.ShapeDtypeStruct((1, D), jnp.float32),
        ],
    )(dp0, dp1, h_train, h_ori)


def _upd_body_first(x, dinv, p0, p1, l0_o, xn_o, y_o):
    t = dinv[...] * (p0[0] + p1[0])
    xn = x[...] - t
    l0_o[...] = x[...] + t
    xn_o[...] = xn
    y_o[...] = dinv[...] * xn


def _upd_body_mid(x, dinv, p0, p1, xn_o, y_o):
    t = dinv[...] * (p0[0] + p1[0])
    xn = x[...] - t
    xn_o[...] = xn
    y_o[...] = dinv[...] * xn


def _upd_body_last(x, dinv, p0, p1, xn_o):
    xn_o[...] = x[...] - dinv[...] * (p0[0] + p1[0])


def _tc_update(x, dinv, p, kind):
    col1 = pl.BlockSpec((B, 1), lambda i: (i, 0))
    full = pl.BlockSpec((B, D), lambda i: (i, 0))
    part0 = pl.BlockSpec((1, B, D), lambda i: (0, i, 0))
    part1 = pl.BlockSpec((1, B, D), lambda i: (1, i, 0))
    nd = jax.ShapeDtypeStruct((N_NODES, D), jnp.float32)
    body, n_out = {
        "first": (_upd_body_first, 3),
        "mid": (_upd_body_mid, 2),
        "last": (_upd_body_last, 1),
    }[kind]
    return pl.pallas_call(
        body,
        grid=(GRID,),
        in_specs=[full, col1, part0, part1],
        out_specs=[full] * n_out,
        out_shape=[nd] * n_out,
    )(x, dinv, p, p)


def _attn_body(hsum, Wq, bq, Wk, Wv, bv, *refs):
    ls = [r[...] for r in refs[:K_HOPS + 1]]
    out = refs[K_HOPS + 1]
    q = (lax.dot_general(hsum[...], Wq[...], (((1,), (1,)), ((), ())),
                         precision=_HI) * (1.0 / N_NODES)) + bq[...]
    wk = lax.dot_general(q, Wk[...], (((1,), (0,)), ((), ())), precision=_HI)
    s = [jnp.sum(l * wk, axis=1, keepdims=True) * _SCALE for l in ls]
    sc = jnp.concatenate(s, axis=1)
    m = jnp.max(sc, axis=1, keepdims=True)
    e = jnp.exp(sc - m)
    a = e / jnp.sum(e, axis=1, keepdims=True)
    comb = a[:, 0:1] * ls[0]
    for k in range(1, K_HOPS + 1):
        comb = comb + a[:, k:k + 1] * ls[k]
    out[...] = lax.dot_general(comb, Wv[...], (((1,), (1,)), ((), ())),
                               precision=_HI) + bv[...]


def _tc_attn(hsum, Wq, bq, Wk, Wv, bv, ls):
    def fix(shape):
        return pl.BlockSpec(shape, lambda i: tuple(0 for _ in shape))

    full = pl.BlockSpec((B, D), lambda i: (i, 0))
    return pl.pallas_call(
        _attn_body,
        grid=(GRID,),
        in_specs=[fix((1, D)), fix((32, D)), fix((1, 32)), fix((32, D)),
                  fix((D, D)), fix((1, D))] + [full] * (K_HOPS + 1),
        out_specs=full,
        out_shape=jax.ShapeDtypeStruct((N_NODES, D), jnp.float32),
    )(hsum, Wq, bq, Wk, Wv, bv, *ls)


def kernel(edge_index, h_train, h_ori, Wq, bq, Wk, bk, Wv, bv):
    del bk
    row = edge_index[0].astype(jnp.int32).reshape(NW, NCH, C)
    col = edge_index[1].astype(jnp.int32).reshape(NW, NCH, C)

    degp = _sc_deg(row)
    dp = degp.reshape(NC, N_PAD, 1)[:, :N_NODES]
    dinv, y, hsum = _tc_prep(dp[0], dp[1], h_train, h_ori)

    x = h_train
    ls = []
    for k in range(K_HOPS):
        p = _sc_hop(y, row, col)
        if k == 0:
            l0, x, y = _tc_update(x, dinv, p, "first")
            ls += [l0, x]
        elif k < K_HOPS - 1:
            x, y = _tc_update(x, dinv, p, "mid")
            ls.append(x)
        else:
            (x,) = _tc_update(x, dinv, p, "last")
            ls.append(x)

    return _tc_attn(hsum, Wq, bq.reshape(1, -1), Wk, Wv, bv.reshape(1, -1),
                    ls)


# speedup vs baseline: 8.9333x; 1.2872x over previous
import math

import jax
import jax.numpy as jnp
from jax import lax
from jax.experimental import pallas as pl
from jax.experimental.pallas import tpu as pltpu
from jax.experimental.pallas import tpu_sc as plsc

N_NODES = 10000
N_PAD = 10240
D = 128
E = 320000
K_HOPS = 8
NC = 2
NS = 16
NW = NC * NS
E_W = E // NW
C = 80
NCH = E_W // C
N_R = 10240
R_T = N_R // NS
D_T = N_PAD // NS

_SCALE = 1.0 / math.sqrt(D)
_HI = lax.Precision.HIGHEST

B = 2000
GRID = N_NODES // B


def _sc_mesh():
    return plsc.VectorSubcoreMesh(core_axis_name="c", subcore_axis_name="s")


def _deg_body(row_hbm, out_hbm, row_v, ones_v, zb, acc1):
    cid = lax.axis_index("c")
    sid = lax.axis_index("s")
    wid = cid * NS + sid

    for i in range(C // 16):
        ones_v[pl.ds(i * 16, 16)] = jnp.ones((16,), jnp.float32)
    for i in range(D_T // 16):
        zb[pl.ds(i * 16, 16)] = jnp.zeros((16,), jnp.float32)
    pltpu.sync_copy(zb, acc1.at[pl.ds(sid * D_T, D_T)])
    pltpu.sync_copy(row_hbm.at[wid], row_v)
    plsc.subcore_barrier()

    def chunk(j, carry):
        pltpu.sync_copy(ones_v, acc1.at[row_v.at[j]], add=True)
        return carry

    lax.fori_loop(0, NCH, chunk, 0)
    plsc.subcore_barrier()
    pltpu.sync_copy(acc1.at[pl.ds(sid * D_T, D_T)],
                    out_hbm.at[cid, pl.ds(sid * D_T, D_T)])


def _sc_deg(row):
    return pl.kernel(
        _deg_body,
        mesh=_sc_mesh(),
        out_type=jax.ShapeDtypeStruct((NC, N_PAD), jnp.float32),
        scratch_types=[
            pltpu.VMEM((NCH, C), jnp.int32),
            pltpu.VMEM((C,), jnp.float32),
            pltpu.VMEM((D_T,), jnp.float32),
            pltpu.VMEM_SHARED((N_PAD,), jnp.float32),
        ],
    )(row)


IB = 25
NB = NCH // IB


def _hop_body(y_hbm, e_hbm, out_hbm, ib, gb0, gb1, acc, sem_i, sem_g0,
              sem_g1):
    cid = lax.axis_index("c")
    sid = lax.axis_index("s")
    wid = cid * NS + sid

    def zi(i, carry):
        r = i // 8
        c0 = (i - r * 8) * 16
        gb0[r, pl.ds(c0, 16)] = jnp.zeros((16,), jnp.float32)
        return carry

    lax.fori_loop(0, C * 8, zi, 0)

    def zc(k, carry):
        pltpu.sync_copy(gb0, acc.at[pl.ds(sid * R_T + k * C, C)])
        return carry

    lax.fori_loop(0, R_T // C, zc, 0)

    pltpu.sync_copy(e_hbm.at[wid, pl.ds(0, IB)], ib.at[0])
    pltpu.async_copy(e_hbm.at[wid, pl.ds(IB, IB)], ib.at[1], sem_i)
    plsc.subcore_barrier()

    def row_of(c):
        return ib.at[(c // IB) % 2, c % IB, 0]

    def col_of(c):
        return ib.at[(c // IB) % 2, c % IB, 1]

    def wait_idx():
        pltpu.make_async_copy(e_hbm.at[0, pl.ds(0, IB)], ib.at[0],
                              sem_i).wait()

    pltpu.async_copy(y_hbm.at[row_of(0)], gb0, sem_g0)

    def do_chunk(c, gb, sem, gb_n, sem_n):
        nxt = c + 1

        @pl.when(jnp.logical_and(nxt % IB == 0, nxt < NCH))
        def _():
            wait_idx()

        pltpu.make_async_copy(y_hbm.at[row_of(c)], gb, sem).wait()

        @pl.when(nxt < NCH)
        def _():
            pltpu.async_copy(y_hbm.at[row_of(nxt)], gb_n, sem_n)

        pltpu.sync_copy(gb, acc.at[col_of(c)], add=True)

        @pl.when(jnp.logical_and(nxt % IB == 0, nxt + IB < NCH))
        def _():
            b = nxt // IB + 1
            start = jnp.minimum(b * IB, NCH - IB)
            pltpu.async_copy(e_hbm.at[wid, pl.ds(start, IB)],
                             ib.at[b % 2], sem_i)

    def pair(j, carry):
        do_chunk(2 * j, gb0, sem_g0, gb1, sem_g1)
        do_chunk(2 * j + 1, gb1, sem_g1, gb0, sem_g0)
        return carry

    lax.fori_loop(0, NCH // 2, pair, 0)
    do_chunk(NCH - 1, gb0, sem_g0, gb1, sem_g1)

    plsc.subcore_barrier()
    pltpu.sync_copy(acc.at[pl.ds(sid * R_T, R_T)],
                    out_hbm.at[cid, pl.ds(sid * R_T, R_T)])


def _sc_hop(y, e):
    return pl.kernel(
        _hop_body,
        mesh=_sc_mesh(),
        out_type=jax.ShapeDtypeStruct((NC, N_R, D), jnp.float32),
        scratch_types=[
            pltpu.VMEM((2, IB, 2, C), jnp.int32),
            pltpu.VMEM((C, D), jnp.float32),
            pltpu.VMEM((C, D), jnp.float32),
            pltpu.VMEM_SHARED((N_R, D), jnp.float32),
            pltpu.SemaphoreType.DMA,
            pltpu.SemaphoreType.DMA,
            pltpu.SemaphoreType.DMA,
        ],
    )(y, e)


def _prep_body(dp0, dp1, ht, ho, dinv_o, y0_o, hsum_o):
    i = pl.program_id(0)
    deg = jnp.maximum(dp0[...] + dp1[...], 1.0)
    dinv = lax.rsqrt(deg)
    dinv_o[...] = dinv
    y0_o[...] = ht[...] * dinv

    @pl.when(i == 0)
    def _():
        hsum_o[...] = jnp.zeros_like(hsum_o)

    hsum_o[...] += jnp.sum(ho[...], axis=0, keepdims=True)


def _tc_prep(dp0, dp1, h_train, h_ori):
    col1 = pl.BlockSpec((B, 1), lambda i: (i, 0))
    full = pl.BlockSpec((B, D), lambda i: (i, 0))
    return pl.pallas_call(
        _prep_body,
        grid=(GRID,),
        in_specs=[col1, col1, full, full],
        out_specs=[col1, full, pl.BlockSpec((1, D), lambda i: (0, 0))],
        out_shape=[
            jax.ShapeDtypeStruct((N_NODES, 1), jnp.float32),
            jax.ShapeDtypeStruct((N_NODES, D), jnp.float32),
            jax.ShapeDtypeStruct((1, D), jnp.float32),
        ],
    )(dp0, dp1, h_train, h_ori)


def _upd_body_first(x, dinv, p0, p1, l0_o, xn_o, y_o):
    t = dinv[...] * (p0[0] + p1[0])
    xn = x[...] - t
    l0_o[...] = x[...] + t
    xn_o[...] = xn
    y_o[...] = dinv[...] * xn


def _upd_body_mid(x, dinv, p0, p1, xn_o, y_o):
    t = dinv[...] * (p0[0] + p1[0])
    xn = x[...] - t
    xn_o[...] = xn
    y_o[...] = dinv[...] * xn


def _upd_body_last(x, dinv, p0, p1, xn_o):
    xn_o[...] = x[...] - dinv[...] * (p0[0] + p1[0])


def _tc_update(x, dinv, p, kind):
    col1 = pl.BlockSpec((B, 1), lambda i: (i, 0))
    full = pl.BlockSpec((B, D), lambda i: (i, 0))
    part0 = pl.BlockSpec((1, B, D), lambda i: (0, i, 0))
    part1 = pl.BlockSpec((1, B, D), lambda i: (1, i, 0))
    nd = jax.ShapeDtypeStruct((N_NODES, D), jnp.float32)
    body, n_out = {
        "first": (_upd_body_first, 3),
        "mid": (_upd_body_mid, 2),
        "last": (_upd_body_last, 1),
    }[kind]
    return pl.pallas_call(
        body,
        grid=(GRID,),
        in_specs=[full, col1, part0, part1],
        out_specs=[full] * n_out,
        out_shape=[nd] * n_out,
    )(x, dinv, p, p)


def _attn_body(hsum, Wq, bq, Wk, Wv, bv, *refs):
    ls = [r[...] for r in refs[:K_HOPS + 1]]
    out = refs[K_HOPS + 1]
    q = (lax.dot_general(hsum[...], Wq[...], (((1,), (1,)), ((), ())),
                         precision=_HI) * (1.0 / N_NODES)) + bq[...]
    wk = lax.dot_general(q, Wk[...], (((1,), (0,)), ((), ())), precision=_HI)
    s = [jnp.sum(l * wk, axis=1, keepdims=True) * _SCALE for l in ls]
    sc = jnp.concatenate(s, axis=1)
    m = jnp.max(sc, axis=1, keepdims=True)
    e = jnp.exp(sc - m)
    a = e / jnp.sum(e, axis=1, keepdims=True)
    comb = a[:, 0:1] * ls[0]
    for k in range(1, K_HOPS + 1):
        comb = comb + a[:, k:k + 1] * ls[k]
    out[...] = lax.dot_general(comb, Wv[...], (((1,), (1,)), ((), ())),
                               precision=_HI) + bv[...]


def _tc_attn(hsum, Wq, bq, Wk, Wv, bv, ls):
    def fix(shape):
        return pl.BlockSpec(shape, lambda i: tuple(0 for _ in shape))

    full = pl.BlockSpec((B, D), lambda i: (i, 0))
    return pl.pallas_call(
        _attn_body,
        grid=(GRID,),
        in_specs=[fix((1, D)), fix((32, D)), fix((1, 32)), fix((32, D)),
                  fix((D, D)), fix((1, D))] + [full] * (K_HOPS + 1),
        out_specs=full,
        out_shape=jax.ShapeDtypeStruct((N_NODES, D), jnp.float32),
    )(hsum, Wq, bq, Wk, Wv, bv, *ls)


def kernel(edge_index, h_train, h_ori, Wq, bq, Wk, bk, Wv, bv):
    del bk
    row = edge_index[0].astype(jnp.int32).reshape(NW, NCH, C)
    col = edge_index[1].astype(jnp.int32).reshape(NW, NCH, C)
    e = jnp.stack([row, col], axis=2)

    degp = _sc_deg(row)
    dp = degp.reshape(NC, N_PAD, 1)[:, :N_NODES]
    dinv, y, hsum = _tc_prep(dp[0], dp[1], h_train, h_ori)

    x = h_train
    ls = []
    for k in range(K_HOPS):
        p = _sc_hop(y, e)
        if k == 0:
            l0, x, y = _tc_update(x, dinv, p, "first")
            ls += [l0, x]
        elif k < K_HOPS - 1:
            x, y = _tc_update(x, dinv, p, "mid")
            ls.append(x)
        else:
            (x,) = _tc_update(x, dinv, p, "last")
            ls.append(x)

    return _tc_attn(hsum, Wq, bq.reshape(1, -1), Wk, Wv, bv.reshape(1, -1),
                    ls)


# speedup vs baseline: 11.2023x; 1.2540x over previous
import math

import jax
import jax.numpy as jnp
from jax import lax
from jax.experimental import pallas as pl
from jax.experimental.pallas import tpu as pltpu
from jax.experimental.pallas import tpu_sc as plsc

N_NODES = 10000
N_PAD = 10240
D = 128
E = 320000
K_HOPS = 8
NC = 2
NS = 16
NW = NC * NS
E_W = E // NW
C = 80
NCH = E_W // C
N_R = 10240
R_T = N_R // NS
D_T = N_PAD // NS

_SCALE = 1.0 / math.sqrt(D)
_HI = lax.Precision.HIGHEST

B = 2000
GRID = N_NODES // B


def _sc_mesh():
    return plsc.VectorSubcoreMesh(core_axis_name="c", subcore_axis_name="s")


def _deg_body(row_hbm, out_hbm, row_v, ones_v, zb, acc1):
    cid = lax.axis_index("c")
    sid = lax.axis_index("s")
    wid = cid * NS + sid

    for i in range(C // 16):
        ones_v[pl.ds(i * 16, 16)] = jnp.ones((16,), jnp.float32)
    for i in range(D_T // 16):
        zb[pl.ds(i * 16, 16)] = jnp.zeros((16,), jnp.float32)
    pltpu.sync_copy(zb, acc1.at[pl.ds(sid * D_T, D_T)])
    pltpu.sync_copy(row_hbm.at[wid], row_v)
    plsc.subcore_barrier()

    def chunk(j, carry):
        pltpu.sync_copy(ones_v, acc1.at[row_v.at[j]], add=True)
        return carry

    lax.fori_loop(0, NCH, chunk, 0)
    plsc.subcore_barrier()
    pltpu.sync_copy(acc1.at[pl.ds(sid * D_T, D_T)],
                    out_hbm.at[cid, pl.ds(sid * D_T, D_T)])


def _sc_deg(row):
    return pl.kernel(
        _deg_body,
        mesh=_sc_mesh(),
        out_type=jax.ShapeDtypeStruct((NC, N_PAD), jnp.float32),
        scratch_types=[
            pltpu.VMEM((NCH, C), jnp.int32),
            pltpu.VMEM((C,), jnp.float32),
            pltpu.VMEM((D_T,), jnp.float32),
            pltpu.VMEM_SHARED((N_PAD,), jnp.float32),
        ],
    )(row)


IB = 25
NB = NCH // IB


def _hop_body(y_hbm, e_hbm, out_hbm, ib, gb0, gb1, acc, sem_i, sem_g0,
              sem_g1):
    cid = lax.axis_index("c")
    sid = lax.axis_index("s")
    wid = cid * NS + sid

    def zi(i, carry):
        r = i // 8
        c0 = (i - r * 8) * 16
        gb0[r, pl.ds(c0, 16)] = jnp.zeros((16,), jnp.float32)
        return carry

    lax.fori_loop(0, C * 8, zi, 0)

    def zc(k, carry):
        pltpu.sync_copy(gb0, acc.at[pl.ds(sid * R_T + k * C, C)])
        return carry

    lax.fori_loop(0, R_T // C, zc, 0)

    pltpu.sync_copy(e_hbm.at[wid, pl.ds(0, IB)], ib.at[0])
    pltpu.async_copy(e_hbm.at[wid, pl.ds(IB, IB)], ib.at[1], sem_i)
    plsc.subcore_barrier()

    def row_of(c):
        return ib.at[(c // IB) % 2, c % IB, 0]

    def col_of(c):
        return ib.at[(c // IB) % 2, c % IB, 1]

    def wait_idx():
        pltpu.make_async_copy(e_hbm.at[0, pl.ds(0, IB)], ib.at[0],
                              sem_i).wait()

    pltpu.async_copy(y_hbm.at[row_of(0)], gb0, sem_g0)

    def do_chunk(c, gb, sem, gb_n, sem_n):
        nxt = c + 1

        @pl.when(jnp.logical_and(nxt % IB == 0, nxt < NCH))
        def _():
            wait_idx()

        @pl.when(nxt < NCH)
        def _():
            pltpu.async_copy(y_hbm.at[row_of(nxt)], gb_n, sem_n)

        pltpu.make_async_copy(y_hbm.at[row_of(c)], gb, sem).wait()
        pltpu.sync_copy(gb, acc.at[col_of(c)], add=True)

        @pl.when(jnp.logical_and(nxt % IB == 0, nxt + IB < NCH))
        def _():
            b = nxt // IB + 1
            start = jnp.minimum(b * IB, NCH - IB)
            pltpu.async_copy(e_hbm.at[wid, pl.ds(start, IB)],
                             ib.at[b % 2], sem_i)

    def pair(j, carry):
        do_chunk(2 * j, gb0, sem_g0, gb1, sem_g1)
        do_chunk(2 * j + 1, gb1, sem_g1, gb0, sem_g0)
        return carry

    lax.fori_loop(0, NCH // 2, pair, 0)
    do_chunk(NCH - 1, gb0, sem_g0, gb1, sem_g1)

    plsc.subcore_barrier()
    pltpu.sync_copy(acc.at[pl.ds(sid * R_T, R_T)],
                    out_hbm.at[cid, pl.ds(sid * R_T, R_T)])


def _sc_hop(y, e):
    return pl.kernel(
        _hop_body,
        mesh=_sc_mesh(),
        out_type=jax.ShapeDtypeStruct((NC, N_R, D), jnp.float32),
        scratch_types=[
            pltpu.VMEM((2, IB, 2, C), jnp.int32),
            pltpu.VMEM((C, D), jnp.float32),
            pltpu.VMEM((C, D), jnp.float32),
            pltpu.VMEM_SHARED((N_R, D), jnp.float32),
            pltpu.SemaphoreType.DMA,
            pltpu.SemaphoreType.DMA,
            pltpu.SemaphoreType.DMA,
        ],
    )(y, e)


def _prep_body(dp0, dp1, ht, ho, dinv_o, y0_o, hsum_o):
    i = pl.program_id(0)
    deg = jnp.maximum(dp0[...] + dp1[...], 1.0)
    dinv = lax.rsqrt(deg)
    dinv_o[...] = dinv
    y0_o[...] = ht[...] * dinv

    @pl.when(i == 0)
    def _():
        hsum_o[...] = jnp.zeros_like(hsum_o)

    hsum_o[...] += jnp.sum(ho[...], axis=0, keepdims=True)


def _tc_prep(dp0, dp1, h_train, h_ori):
    col1 = pl.BlockSpec((B, 1), lambda i: (i, 0))
    full = pl.BlockSpec((B, D), lambda i: (i, 0))
    return pl.pallas_call(
        _prep_body,
        grid=(GRID,),
        in_specs=[col1, col1, full, full],
        out_specs=[col1, full, pl.BlockSpec((1, D), lambda i: (0, 0))],
        out_shape=[
            jax.ShapeDtypeStruct((N_NODES, 1), jnp.float32),
            jax.ShapeDtypeStruct((N_NODES, D), jnp.float32),
            jax.ShapeDtypeStruct((1, D), jnp.float32),
        ],
    )(dp0, dp1, h_train, h_ori)


def _upd_body_first(x, dinv, p0, p1, l0_o, xn_o, y_o):
    t = dinv[...] * (p0[0] + p1[0])
    xn = x[...] - t
    l0_o[...] = x[...] + t
    xn_o[...] = xn
    y_o[...] = dinv[...] * xn


def _upd_body_mid(x, dinv, p0, p1, xn_o, y_o):
    t = dinv[...] * (p0[0] + p1[0])
    xn = x[...] - t
    xn_o[...] = xn
    y_o[...] = dinv[...] * xn


def _upd_body_last(x, dinv, p0, p1, xn_o):
    xn_o[...] = x[...] - dinv[...] * (p0[0] + p1[0])


def _tc_update(x, dinv, p, kind):
    col1 = pl.BlockSpec((B, 1), lambda i: (i, 0))
    full = pl.BlockSpec((B, D), lambda i: (i, 0))
    part0 = pl.BlockSpec((1, B, D), lambda i: (0, i, 0))
    part1 = pl.BlockSpec((1, B, D), lambda i: (1, i, 0))
    nd = jax.ShapeDtypeStruct((N_NODES, D), jnp.float32)
    body, n_out = {
        "first": (_upd_body_first, 3),
        "mid": (_upd_body_mid, 2),
        "last": (_upd_body_last, 1),
    }[kind]
    return pl.pallas_call(
        body,
        grid=(GRID,),
        in_specs=[full, col1, part0, part1],
        out_specs=[full] * n_out,
        out_shape=[nd] * n_out,
    )(x, dinv, p, p)


def _attn_body(hsum, Wq, bq, Wk, Wv, bv, *refs):
    ls = [r[...] for r in refs[:K_HOPS + 1]]
    out = refs[K_HOPS + 1]
    q = (lax.dot_general(hsum[...], Wq[...], (((1,), (1,)), ((), ())),
                         precision=_HI) * (1.0 / N_NODES)) + bq[...]
    wk = lax.dot_general(q, Wk[...], (((1,), (0,)), ((), ())), precision=_HI)
    s = [jnp.sum(l * wk, axis=1, keepdims=True) * _SCALE for l in ls]
    sc = jnp.concatenate(s, axis=1)
    m = jnp.max(sc, axis=1, keepdims=True)
    e = jnp.exp(sc - m)
    a = e / jnp.sum(e, axis=1, keepdims=True)
    comb = a[:, 0:1] * ls[0]
    for k in range(1, K_HOPS + 1):
        comb = comb + a[:, k:k + 1] * ls[k]
    out[...] = lax.dot_general(comb, Wv[...], (((1,), (1,)), ((), ())),
                               precision=_HI) + bv[...]


def _tc_attn(hsum, Wq, bq, Wk, Wv, bv, ls):
    def fix(shape):
        return pl.BlockSpec(shape, lambda i: tuple(0 for _ in shape))

    full = pl.BlockSpec((B, D), lambda i: (i, 0))
    return pl.pallas_call(
        _attn_body,
        grid=(GRID,),
        in_specs=[fix((1, D)), fix((32, D)), fix((1, 32)), fix((32, D)),
                  fix((D, D)), fix((1, D))] + [full] * (K_HOPS + 1),
        out_specs=full,
        out_shape=jax.ShapeDtypeStruct((N_NODES, D), jnp.float32),
    )(hsum, Wq, bq, Wk, Wv, bv, *ls)


def kernel(edge_index, h_train, h_ori, Wq, bq, Wk, bk, Wv, bv):
    del bk
    row = edge_index[0].astype(jnp.int32).reshape(NW, NCH, C)
    col = edge_index[1].astype(jnp.int32).reshape(NW, NCH, C)
    e = jnp.stack([row, col], axis=2)

    degp = _sc_deg(row)
    dp = degp.reshape(NC, N_PAD, 1)[:, :N_NODES]
    dinv, y, hsum = _tc_prep(dp[0], dp[1], h_train, h_ori)

    x = h_train
    ls = []
    for k in range(K_HOPS):
        p = _sc_hop(y, e)
        if k == 0:
            l0, x, y = _tc_update(x, dinv, p, "first")
            ls += [l0, x]
        elif k < K_HOPS - 1:
            x, y = _tc_update(x, dinv, p, "mid")
            ls.append(x)
        else:
            (x,) = _tc_update(x, dinv, p, "last")
            ls.append(x)

    return _tc_attn(hsum, Wq, bq.reshape(1, -1), Wk, Wv, bv.reshape(1, -1),
                    ls)


# speedup vs baseline: 13.2617x; 1.1838x over previous
import math

import jax
import jax.numpy as jnp
from jax import lax
from jax.experimental import pallas as pl
from jax.experimental.pallas import tpu as pltpu
from jax.experimental.pallas import tpu_sc as plsc

N_NODES = 10000
N_PAD = 10240
D = 128
E = 320000
K_HOPS = 8
NC = 2
NS = 16
NW = NC * NS
E_W = E // NW
C = 80
NCH = E_W // C
N_R = 10240
R_T = N_R // NS
D_T = N_PAD // NS

_SCALE = 1.0 / math.sqrt(D)
_HI = lax.Precision.HIGHEST

B = 2000
GRID = N_NODES // B


def _sc_mesh():
    return plsc.VectorSubcoreMesh(core_axis_name="c", subcore_axis_name="s")


def _deg_body(row_hbm, out_hbm, row_v, ones_v, zb, acc1):
    cid = lax.axis_index("c")
    sid = lax.axis_index("s")
    wid = cid * NS + sid

    for i in range(C // 16):
        ones_v[pl.ds(i * 16, 16)] = jnp.ones((16,), jnp.float32)
    for i in range(D_T // 16):
        zb[pl.ds(i * 16, 16)] = jnp.zeros((16,), jnp.float32)
    pltpu.sync_copy(zb, acc1.at[pl.ds(sid * D_T, D_T)])
    pltpu.sync_copy(row_hbm.at[wid], row_v)
    plsc.subcore_barrier()

    def chunk(j, carry):
        pltpu.sync_copy(ones_v, acc1.at[row_v.at[j]], add=True)
        return carry

    lax.fori_loop(0, NCH, chunk, 0)
    plsc.subcore_barrier()
    pltpu.sync_copy(acc1.at[pl.ds(sid * D_T, D_T)],
                    out_hbm.at[cid, pl.ds(sid * D_T, D_T)])


def _sc_deg(row):
    return pl.kernel(
        _deg_body,
        mesh=_sc_mesh(),
        out_type=jax.ShapeDtypeStruct((NC, N_PAD), jnp.float32),
        scratch_types=[
            pltpu.VMEM((NCH, C), jnp.int32),
            pltpu.VMEM((C,), jnp.float32),
            pltpu.VMEM((D_T,), jnp.float32),
            pltpu.VMEM_SHARED((N_PAD,), jnp.float32),
        ],
    )(row)


IB = 5
NB = NCH // IB


def _hop_body(y_hbm, e_hbm, out_hbm, ib, gb0, gb1, gb2, acc, sem_i, sem_g0,
              sem_g1, sem_g2):
    cid = lax.axis_index("c")
    sid = lax.axis_index("s")
    wid = cid * NS + sid

    def zi(i, carry):
        r = i // 8
        c0 = (i - r * 8) * 16
        gb0[r, pl.ds(c0, 16)] = jnp.zeros((16,), jnp.float32)
        return carry

    lax.fori_loop(0, C * 8, zi, 0)

    def zc(k, carry):
        pltpu.sync_copy(gb0, acc.at[pl.ds(sid * R_T + k * C, C)])
        return carry

    lax.fori_loop(0, R_T // C, zc, 0)

    pltpu.sync_copy(e_hbm.at[wid, pl.ds(0, IB)], ib.at[0])
    pltpu.async_copy(e_hbm.at[wid, pl.ds(IB, IB)], ib.at[1], sem_i)
    plsc.subcore_barrier()

    def row_of(c):
        return ib.at[(c // IB) % 3, c % IB, 0]

    def col_of(c):
        return ib.at[(c // IB) % 3, c % IB, 1]

    pltpu.async_copy(y_hbm.at[row_of(0)], gb0, sem_g0)
    pltpu.async_copy(y_hbm.at[row_of(1)], gb1, sem_g1)

    def do_chunk(c, gb, sem_g, gb_2, sem_g_2):
        nx2 = c + 2

        @pl.when(jnp.logical_and(nx2 % IB == 0, nx2 < NCH))
        def _():
            pltpu.make_async_copy(e_hbm.at[0, pl.ds(0, IB)], ib.at[0],
                                  sem_i).wait()
            b = nx2 // IB + 1

            @pl.when(b < NB)
            def _():
                start = jnp.minimum(b * IB, NCH - IB)
                pltpu.async_copy(e_hbm.at[wid, pl.ds(start, IB)],
                                 ib.at[b % 3], sem_i)

        @pl.when(nx2 < NCH)
        def _():
            pltpu.async_copy(y_hbm.at[row_of(nx2)], gb_2, sem_g_2)

        pltpu.make_async_copy(y_hbm.at[row_of(c)], gb, sem_g).wait()
        pltpu.sync_copy(gb, acc.at[col_of(c)], add=True)

    def triple(j, carry):
        do_chunk(3 * j, gb0, sem_g0, gb2, sem_g2)
        do_chunk(3 * j + 1, gb1, sem_g1, gb0, sem_g0)
        do_chunk(3 * j + 2, gb2, sem_g2, gb1, sem_g1)
        return carry

    lax.fori_loop(0, NCH // 3, triple, 0)
    do_chunk(NCH - 2, gb0, sem_g0, gb2, sem_g2)
    do_chunk(NCH - 1, gb1, sem_g1, gb0, sem_g0)

    plsc.subcore_barrier()
    pltpu.sync_copy(acc.at[pl.ds(sid * R_T, R_T)],
                    out_hbm.at[cid, pl.ds(sid * R_T, R_T)])


def _sc_hop(y, e):
    return pl.kernel(
        _hop_body,
        mesh=_sc_mesh(),
        out_type=jax.ShapeDtypeStruct((NC, N_R, D), jnp.float32),
        scratch_types=[
            pltpu.VMEM((3, IB, 2, C), jnp.int32),
            pltpu.VMEM((C, D), jnp.float32),
            pltpu.VMEM((C, D), jnp.float32),
            pltpu.VMEM((C, D), jnp.float32),
            pltpu.VMEM_SHARED((N_R, D), jnp.float32),
            pltpu.SemaphoreType.DMA,
            pltpu.SemaphoreType.DMA,
            pltpu.SemaphoreType.DMA,
            pltpu.SemaphoreType.DMA,
        ],
    )(y, e)


def _prep_body(dp0, dp1, ht, ho, dinv_o, y0_o, hsum_o):
    i = pl.program_id(0)
    deg = jnp.maximum(dp0[...] + dp1[...], 1.0)
    dinv = lax.rsqrt(deg)
    dinv_o[...] = dinv
    y0_o[...] = ht[...] * dinv

    @pl.when(i == 0)
    def _():
        hsum_o[...] = jnp.zeros_like(hsum_o)

    hsum_o[...] += jnp.sum(ho[...], axis=0, keepdims=True)


def _tc_prep(dp0, dp1, h_train, h_ori):
    col1 = pl.BlockSpec((B, 1), lambda i: (i, 0))
    full = pl.BlockSpec((B, D), lambda i: (i, 0))
    return pl.pallas_call(
        _prep_body,
        grid=(GRID,),
        in_specs=[col1, col1, full, full],
        out_specs=[col1, full, pl.BlockSpec((1, D), lambda i: (0, 0))],
        out_shape=[
            jax.ShapeDtypeStruct((N_NODES, 1), jnp.float32),
            jax.ShapeDtypeStruct((N_NODES, D), jnp.float32),
            jax.ShapeDtypeStruct((1, D), jnp.float32),
        ],
    )(dp0, dp1, h_train, h_ori)


def _upd_body_first(x, dinv, p0, p1, l0_o, xn_o, y_o):
    t = dinv[...] * (p0[0] + p1[0])
    xn = x[...] - t
    l0_o[...] = x[...] + t
    xn_o[...] = xn
    y_o[...] = dinv[...] * xn


def _upd_body_mid(x, dinv, p0, p1, xn_o, y_o):
    t = dinv[...] * (p0[0] + p1[0])
    xn = x[...] - t
    xn_o[...] = xn
    y_o[...] = dinv[...] * xn


def _upd_body_last(x, dinv, p0, p1, xn_o):
    xn_o[...] = x[...] - dinv[...] * (p0[0] + p1[0])


def _tc_update(x, dinv, p, kind):
    col1 = pl.BlockSpec((B, 1), lambda i: (i, 0))
    full = pl.BlockSpec((B, D), lambda i: (i, 0))
    part0 = pl.BlockSpec((1, B, D), lambda i: (0, i, 0))
    part1 = pl.BlockSpec((1, B, D), lambda i: (1, i, 0))
    nd = jax.ShapeDtypeStruct((N_NODES, D), jnp.float32)
    body, n_out = {
        "first": (_upd_body_first, 3),
        "mid": (_upd_body_mid, 2),
        "last": (_upd_body_last, 1),
    }[kind]
    return pl.pallas_call(
        body,
        grid=(GRID,),
        in_specs=[full, col1, part0, part1],
        out_specs=[full] * n_out,
        out_shape=[nd] * n_out,
    )(x, dinv, p, p)


def _attn_body(hsum, Wq, bq, Wk, Wv, bv, *refs):
    ls = [r[...] for r in refs[:K_HOPS + 1]]
    out = refs[K_HOPS + 1]
    q = (lax.dot_general(hsum[...], Wq[...], (((1,), (1,)), ((), ())),
                         precision=_HI) * (1.0 / N_NODES)) + bq[...]
    wk = lax.dot_general(q, Wk[...], (((1,), (0,)), ((), ())), precision=_HI)
    s = [jnp.sum(l * wk, axis=1, keepdims=True) * _SCALE for l in ls]
    sc = jnp.concatenate(s, axis=1)
    m = jnp.max(sc, axis=1, keepdims=True)
    e = jnp.exp(sc - m)
    a = e / jnp.sum(e, axis=1, keepdims=True)
    comb = a[:, 0:1] * ls[0]
    for k in range(1, K_HOPS + 1):
        comb = comb + a[:, k:k + 1] * ls[k]
    out[...] = lax.dot_general(comb, Wv[...], (((1,), (1,)), ((), ())),
                               precision=_HI) + bv[...]


def _tc_attn(hsum, Wq, bq, Wk, Wv, bv, ls):
    def fix(shape):
        return pl.BlockSpec(shape, lambda i: tuple(0 for _ in shape))

    full = pl.BlockSpec((B, D), lambda i: (i, 0))
    return pl.pallas_call(
        _attn_body,
        grid=(GRID,),
        in_specs=[fix((1, D)), fix((32, D)), fix((1, 32)), fix((32, D)),
                  fix((D, D)), fix((1, D))] + [full] * (K_HOPS + 1),
        out_specs=full,
        out_shape=jax.ShapeDtypeStruct((N_NODES, D), jnp.float32),
    )(hsum, Wq, bq, Wk, Wv, bv, *ls)


def kernel(edge_index, h_train, h_ori, Wq, bq, Wk, bk, Wv, bv):
    del bk
    row = edge_index[0].astype(jnp.int32).reshape(NW, NCH, C)
    col = edge_index[1].astype(jnp.int32).reshape(NW, NCH, C)
    e = jnp.stack([row, col], axis=2)

    degp = _sc_deg(row)
    dp = degp.reshape(NC, N_PAD, 1)[:, :N_NODES]
    dinv, y, hsum = _tc_prep(dp[0], dp[1], h_train, h_ori)

    x = h_train
    ls = []
    for k in range(K_HOPS):
        p = _sc_hop(y, e)
        if k == 0:
            l0, x, y = _tc_update(x, dinv, p, "first")
            ls += [l0, x]
        elif k < K_HOPS - 1:
            x, y = _tc_update(x, dinv, p, "mid")
            ls.append(x)
        else:
            (x,) = _tc_update(x, dinv, p, "last")
            ls.append(x)

    return _tc_attn(hsum, Wq, bq.reshape(1, -1), Wk, Wv, bv.reshape(1, -1),
                    ls)


# speedup vs baseline: 13.2761x; 1.0011x over previous
import math

import jax
import jax.numpy as jnp
from jax import lax
from jax.experimental import pallas as pl
from jax.experimental.pallas import tpu as pltpu
from jax.experimental.pallas import tpu_sc as plsc

N_NODES = 10000
N_PAD = 10240
D = 128
E = 320000
K_HOPS = 8
NC = 2
NS = 16
NW = NC * NS
E_W = E // NW
C = 80
NCH = E_W // C
N_R = 10240
R_T = N_R // NS
D_T = N_PAD // NS

_SCALE = 1.0 / math.sqrt(D)
_HI = lax.Precision.HIGHEST

B = 2000
GRID = N_NODES // B


def _sc_mesh():
    return plsc.VectorSubcoreMesh(core_axis_name="c", subcore_axis_name="s")


def _deg_body(row_hbm, out_hbm, row_v, ones_v, zb, acc1):
    cid = lax.axis_index("c")
    sid = lax.axis_index("s")
    wid = cid * NS + sid

    for i in range(C // 16):
        ones_v[pl.ds(i * 16, 16)] = jnp.ones((16,), jnp.float32)
    for i in range(D_T // 16):
        zb[pl.ds(i * 16, 16)] = jnp.zeros((16,), jnp.float32)
    pltpu.sync_copy(zb, acc1.at[pl.ds(sid * D_T, D_T)])
    pltpu.sync_copy(row_hbm.at[wid], row_v)
    plsc.subcore_barrier()

    def chunk(j, carry):
        pltpu.sync_copy(ones_v, acc1.at[row_v.at[j]], add=True)
        return carry

    lax.fori_loop(0, NCH, chunk, 0)
    plsc.subcore_barrier()
    pltpu.sync_copy(acc1.at[pl.ds(sid * D_T, D_T)],
                    out_hbm.at[cid, pl.ds(sid * D_T, D_T)])


def _sc_deg(row):
    return pl.kernel(
        _deg_body,
        mesh=_sc_mesh(),
        out_type=jax.ShapeDtypeStruct((NC, N_PAD), jnp.float32),
        scratch_types=[
            pltpu.VMEM((NCH, C), jnp.int32),
            pltpu.VMEM((C,), jnp.float32),
            pltpu.VMEM((D_T,), jnp.float32),
            pltpu.VMEM_SHARED((N_PAD,), jnp.float32),
        ],
    )(row)


IB = 5
NB = NCH // IB


def _hop_body(y_hbm, e_hbm, out_hbm, ib, gb0, gb1, gb2, acc, sem_i, sem_g0,
              sem_g1, sem_g2, sem_s0, sem_s1, sem_s2):
    cid = lax.axis_index("c")
    sid = lax.axis_index("s")
    wid = cid * NS + sid

    def zi(i, carry):
        r = i // 8
        c0 = (i - r * 8) * 16
        gb0[r, pl.ds(c0, 16)] = jnp.zeros((16,), jnp.float32)
        return carry

    lax.fori_loop(0, C * 8, zi, 0)

    def zc(k, carry):
        pltpu.sync_copy(gb0, acc.at[pl.ds(sid * R_T + k * C, C)])
        return carry

    lax.fori_loop(0, R_T // C, zc, 0)

    pltpu.sync_copy(e_hbm.at[wid, pl.ds(0, IB)], ib.at[0])
    pltpu.async_copy(e_hbm.at[wid, pl.ds(IB, IB)], ib.at[1], sem_i)
    plsc.subcore_barrier()

    def row_of(c):
        return ib.at[(c // IB) % 3, c % IB, 0]

    def col_of(c):
        return ib.at[(c // IB) % 3, c % IB, 1]

    pltpu.async_copy(y_hbm.at[row_of(0)], gb0, sem_g0)
    pltpu.async_copy(y_hbm.at[row_of(1)], gb1, sem_g1)

    def do_chunk(c, gb, sem_g, sem_s, gb_2, sem_g_2, sem_s_2):
        nx2 = c + 2

        @pl.when(jnp.logical_and(nx2 % IB == 0, nx2 < NCH))
        def _():
            pltpu.make_async_copy(e_hbm.at[0, pl.ds(0, IB)], ib.at[0],
                                  sem_i).wait()
            b = nx2 // IB + 1

            @pl.when(b < NB)
            def _():
                start = jnp.minimum(b * IB, NCH - IB)
                pltpu.async_copy(e_hbm.at[wid, pl.ds(start, IB)],
                                 ib.at[b % 3], sem_i)

        @pl.when(c >= 1)
        def _():
            cp = jnp.maximum(c - 1, 0)
            pltpu.make_async_copy(gb_2, acc.at[col_of(cp)], sem_s_2).wait()

        @pl.when(nx2 < NCH)
        def _():
            pltpu.async_copy(y_hbm.at[row_of(nx2)], gb_2, sem_g_2)

        pltpu.make_async_copy(y_hbm.at[row_of(c)], gb, sem_g).wait()
        pltpu.async_copy(gb, acc.at[col_of(c)], sem_s, add=True)

    def triple(j, carry):
        do_chunk(3 * j, gb0, sem_g0, sem_s0, gb2, sem_g2, sem_s2)
        do_chunk(3 * j + 1, gb1, sem_g1, sem_s1, gb0, sem_g0, sem_s0)
        do_chunk(3 * j + 2, gb2, sem_g2, sem_s2, gb1, sem_g1, sem_s1)
        return carry

    lax.fori_loop(0, NCH // 3, triple, 0)
    do_chunk(NCH - 2, gb0, sem_g0, sem_s0, gb2, sem_g2, sem_s2)
    do_chunk(NCH - 1, gb1, sem_g1, sem_s1, gb0, sem_g0, sem_s0)
    pltpu.make_async_copy(gb1, acc.at[col_of(NCH - 1)], sem_s1).wait()

    plsc.subcore_barrier()
    pltpu.sync_copy(acc.at[pl.ds(sid * R_T, R_T)],
                    out_hbm.at[cid, pl.ds(sid * R_T, R_T)])


def _sc_hop(y, e):
    return pl.kernel(
        _hop_body,
        mesh=_sc_mesh(),
        out_type=jax.ShapeDtypeStruct((NC, N_R, D), jnp.float32),
        scratch_types=[
            pltpu.VMEM((3, IB, 2, C), jnp.int32),
            pltpu.VMEM((C, D), jnp.float32),
            pltpu.VMEM((C, D), jnp.float32),
            pltpu.VMEM((C, D), jnp.float32),
            pltpu.VMEM_SHARED((N_R, D), jnp.float32),
            pltpu.SemaphoreType.DMA,
            pltpu.SemaphoreType.DMA,
            pltpu.SemaphoreType.DMA,
            pltpu.SemaphoreType.DMA,
            pltpu.SemaphoreType.DMA,
            pltpu.SemaphoreType.DMA,
            pltpu.SemaphoreType.DMA,
        ],
    )(y, e)


def _prep_body(dp0, dp1, ht, ho, dinv_o, y0_o, hsum_o):
    i = pl.program_id(0)
    deg = jnp.maximum(dp0[...] + dp1[...], 1.0)
    dinv = lax.rsqrt(deg)
    dinv_o[...] = dinv
    y0_o[...] = ht[...] * dinv

    @pl.when(i == 0)
    def _():
        hsum_o[...] = jnp.zeros_like(hsum_o)

    hsum_o[...] += jnp.sum(ho[...], axis=0, keepdims=True)


def _tc_prep(dp0, dp1, h_train, h_ori):
    col1 = pl.BlockSpec((B, 1), lambda i: (i, 0))
    full = pl.BlockSpec((B, D), lambda i: (i, 0))
    return pl.pallas_call(
        _prep_body,
        grid=(GRID,),
        in_specs=[col1, col1, full, full],
        out_specs=[col1, full, pl.BlockSpec((1, D), lambda i: (0, 0))],
        out_shape=[
            jax.ShapeDtypeStruct((N_NODES, 1), jnp.float32),
            jax.ShapeDtypeStruct((N_NODES, D), jnp.float32),
            jax.ShapeDtypeStruct((1, D), jnp.float32),
        ],
    )(dp0, dp1, h_train, h_ori)


def _upd_body_first(x, dinv, p0, p1, l0_o, xn_o, y_o):
    t = dinv[...] * (p0[0] + p1[0])
    xn = x[...] - t
    l0_o[...] = x[...] + t
    xn_o[...] = xn
    y_o[...] = dinv[...] * xn


def _upd_body_mid(x, dinv, p0, p1, xn_o, y_o):
    t = dinv[...] * (p0[0] + p1[0])
    xn = x[...] - t
    xn_o[...] = xn
    y_o[...] = dinv[...] * xn


def _upd_body_last(x, dinv, p0, p1, xn_o):
    xn_o[...] = x[...] - dinv[...] * (p0[0] + p1[0])


def _tc_update(x, dinv, p, kind):
    col1 = pl.BlockSpec((B, 1), lambda i: (i, 0))
    full = pl.BlockSpec((B, D), lambda i: (i, 0))
    part0 = pl.BlockSpec((1, B, D), lambda i: (0, i, 0))
    part1 = pl.BlockSpec((1, B, D), lambda i: (1, i, 0))
    nd = jax.ShapeDtypeStruct((N_NODES, D), jnp.float32)
    body, n_out = {
        "first": (_upd_body_first, 3),
        "mid": (_upd_body_mid, 2),
        "last": (_upd_body_last, 1),
    }[kind]
    return pl.pallas_call(
        body,
        grid=(GRID,),
        in_specs=[full, col1, part0, part1],
        out_specs=[full] * n_out,
        out_shape=[nd] * n_out,
    )(x, dinv, p, p)


def _attn_body(hsum, Wq, bq, Wk, Wv, bv, *refs):
    ls = [r[...] for r in refs[:K_HOPS + 1]]
    out = refs[K_HOPS + 1]
    q = (lax.dot_general(hsum[...], Wq[...], (((1,), (1,)), ((), ())),
                         precision=_HI) * (1.0 / N_NODES)) + bq[...]
    wk = lax.dot_general(q, Wk[...], (((1,), (0,)), ((), ())), precision=_HI)
    s = [jnp.sum(l * wk, axis=1, keepdims=True) * _SCALE for l in ls]
    sc = jnp.concatenate(s, axis=1)
    m = jnp.max(sc, axis=1, keepdims=True)
    e = jnp.exp(sc - m)
    a = e / jnp.sum(e, axis=1, keepdims=True)
    comb = a[:, 0:1] * ls[0]
    for k in range(1, K_HOPS + 1):
        comb = comb + a[:, k:k + 1] * ls[k]
    out[...] = lax.dot_general(comb, Wv[...], (((1,), (1,)), ((), ())),
                               precision=_HI) + bv[...]


def _tc_attn(hsum, Wq, bq, Wk, Wv, bv, ls):
    def fix(shape):
        return pl.BlockSpec(shape, lambda i: tuple(0 for _ in shape))

    full = pl.BlockSpec((B, D), lambda i: (i, 0))
    return pl.pallas_call(
        _attn_body,
        grid=(GRID,),
        in_specs=[fix((1, D)), fix((32, D)), fix((1, 32)), fix((32, D)),
                  fix((D, D)), fix((1, D))] + [full] * (K_HOPS + 1),
        out_specs=full,
        out_shape=jax.ShapeDtypeStruct((N_NODES, D), jnp.float32),
    )(hsum, Wq, bq, Wk, Wv, bv, *ls)


def kernel(edge_index, h_train, h_ori, Wq, bq, Wk, bk, Wv, bv):
    del bk
    row = edge_index[0].astype(jnp.int32).reshape(NW, NCH, C)
    col = edge_index[1].astype(jnp.int32).reshape(NW, NCH, C)
    e = jnp.stack([row, col], axis=2)

    degp = _sc_deg(row)
    dp = degp.reshape(NC, N_PAD, 1)[:, :N_NODES]
    dinv, y, hsum = _tc_prep(dp[0], dp[1], h_train, h_ori)

    x = h_train
    ls = []
    for k in range(K_HOPS):
        p = _sc_hop(y, e)
        if k == 0:
            l0, x, y = _tc_update(x, dinv, p, "first")
            ls += [l0, x]
        elif k < K_HOPS - 1:
            x, y = _tc_update(x, dinv, p, "mid")
            ls.append(x)
        else:
            (x,) = _tc_update(x, dinv, p, "last")
            ls.append(x)

    return _tc_attn(hsum, Wq, bq.reshape(1, -1), Wk, Wv, bv.reshape(1, -1),
                    ls)


# speedup vs baseline: 13.2957x; 1.0015x over previous
import math

import jax
import jax.numpy as jnp
from jax import lax
from jax.experimental import pallas as pl
from jax.experimental.pallas import tpu as pltpu
from jax.experimental.pallas import tpu_sc as plsc

N_NODES = 10000
N_PAD = 10240
D = 128
E = 320000
K_HOPS = 8
NC = 2
NS = 16
NW = NC * NS
E_W = E // NW
C = 80
NCH = E_W // C
N_R = 10240
R_T = N_R // NS
D_T = N_PAD // NS

_SCALE = 1.0 / math.sqrt(D)
_HI = lax.Precision.HIGHEST

B = 2000
GRID = N_NODES // B


def _sc_mesh():
    return plsc.VectorSubcoreMesh(core_axis_name="c", subcore_axis_name="s")


def _deg_body(row_hbm, out_hbm, row_v, ones_v, zb, acc1, sem):
    cid = lax.axis_index("c")
    sid = lax.axis_index("s")
    wid = cid * NS + sid

    for i in range(C // 16):
        ones_v[pl.ds(i * 16, 16)] = jnp.ones((16,), jnp.float32)
    for i in range(D_T // 16):
        zb[pl.ds(i * 16, 16)] = jnp.zeros((16,), jnp.float32)
    pltpu.sync_copy(zb, acc1.at[pl.ds(sid * D_T, D_T)])
    pltpu.sync_copy(row_hbm.at[wid], row_v)
    plsc.subcore_barrier()

    def iss(j, carry):
        pltpu.async_copy(ones_v, acc1.at[row_v.at[j]], sem, add=True)
        return carry

    lax.fori_loop(0, NCH, iss, 0)

    def drain(j, carry):
        pltpu.make_async_copy(ones_v, acc1.at[row_v.at[j]], sem).wait()
        return carry

    lax.fori_loop(0, NCH, drain, 0)
    plsc.subcore_barrier()
    pltpu.sync_copy(acc1.at[pl.ds(sid * D_T, D_T)],
                    out_hbm.at[cid, pl.ds(sid * D_T, D_T)])


def _sc_deg(row):
    return pl.kernel(
        _deg_body,
        mesh=_sc_mesh(),
        out_type=jax.ShapeDtypeStruct((NC, N_PAD), jnp.float32),
        scratch_types=[
            pltpu.VMEM((NCH, C), jnp.int32),
            pltpu.VMEM((C,), jnp.float32),
            pltpu.VMEM((D_T,), jnp.float32),
            pltpu.VMEM_SHARED((N_PAD,), jnp.float32),
            pltpu.SemaphoreType.DMA,
        ],
    )(row)


IB = 5
NB = NCH // IB


def _hop_body(y_hbm, e_hbm, out_hbm, ib, gb0, gb1, gb2, acc, sem_i, sem_g0,
              sem_g1, sem_g2, sem_s0, sem_s1, sem_s2):
    cid = lax.axis_index("c")
    sid = lax.axis_index("s")
    wid = cid * NS + sid

    def zi(i, carry):
        r = i // 8
        c0 = (i - r * 8) * 16
        gb0[r, pl.ds(c0, 16)] = jnp.zeros((16,), jnp.float32)
        return carry

    lax.fori_loop(0, C * 8, zi, 0)

    def zc(k, carry):
        pltpu.sync_copy(gb0, acc.at[pl.ds(sid * R_T + k * C, C)])
        return carry

    lax.fori_loop(0, R_T // C, zc, 0)

    pltpu.sync_copy(e_hbm.at[wid, pl.ds(0, IB)], ib.at[0])
    pltpu.async_copy(e_hbm.at[wid, pl.ds(IB, IB)], ib.at[1], sem_i)
    plsc.subcore_barrier()

    def row_of(c):
        return ib.at[(c // IB) % 3, c % IB, 0]

    def col_of(c):
        return ib.at[(c // IB) % 3, c % IB, 1]

    pltpu.async_copy(y_hbm.at[row_of(0)], gb0, sem_g0)
    pltpu.async_copy(y_hbm.at[row_of(1)], gb1, sem_g1)

    def do_chunk(c, gb, sem_g, sem_s, gb_2, sem_g_2, sem_s_2):
        nx2 = c + 2

        @pl.when(jnp.logical_and(nx2 % IB == 0, nx2 < NCH))
        def _():
            pltpu.make_async_copy(e_hbm.at[0, pl.ds(0, IB)], ib.at[0],
                                  sem_i).wait()
            b = nx2 // IB + 1

            @pl.when(b < NB)
            def _():
                start = jnp.minimum(b * IB, NCH - IB)
                pltpu.async_copy(e_hbm.at[wid, pl.ds(start, IB)],
                                 ib.at[b % 3], sem_i)

        @pl.when(c >= 1)
        def _():
            cp = jnp.maximum(c - 1, 0)
            pltpu.make_async_copy(gb_2, acc.at[col_of(cp)], sem_s_2).wait()

        @pl.when(nx2 < NCH)
        def _():
            pltpu.async_copy(y_hbm.at[row_of(nx2)], gb_2, sem_g_2)

        pltpu.make_async_copy(y_hbm.at[row_of(c)], gb, sem_g).wait()
        pltpu.async_copy(gb, acc.at[col_of(c)], sem_s, add=True)

    def triple(j, carry):
        do_chunk(3 * j, gb0, sem_g0, sem_s0, gb2, sem_g2, sem_s2)
        do_chunk(3 * j + 1, gb1, sem_g1, sem_s1, gb0, sem_g0, sem_s0)
        do_chunk(3 * j + 2, gb2, sem_g2, sem_s2, gb1, sem_g1, sem_s1)
        return carry

    lax.fori_loop(0, NCH // 3, triple, 0)
    do_chunk(NCH - 2, gb0, sem_g0, sem_s0, gb2, sem_g2, sem_s2)
    do_chunk(NCH - 1, gb1, sem_g1, sem_s1, gb0, sem_g0, sem_s0)
    pltpu.make_async_copy(gb1, acc.at[col_of(NCH - 1)], sem_s1).wait()

    plsc.subcore_barrier()
    pltpu.sync_copy(acc.at[pl.ds(sid * R_T, R_T)],
                    out_hbm.at[cid, pl.ds(sid * R_T, R_T)])


def _sc_hop(y, e):
    return pl.kernel(
        _hop_body,
        mesh=_sc_mesh(),
        out_type=jax.ShapeDtypeStruct((NC, N_R, D), jnp.float32),
        scratch_types=[
            pltpu.VMEM((3, IB, 2, C), jnp.int32),
            pltpu.VMEM((C, D), jnp.float32),
            pltpu.VMEM((C, D), jnp.float32),
            pltpu.VMEM((C, D), jnp.float32),
            pltpu.VMEM_SHARED((N_R, D), jnp.float32),
            pltpu.SemaphoreType.DMA,
            pltpu.SemaphoreType.DMA,
            pltpu.SemaphoreType.DMA,
            pltpu.SemaphoreType.DMA,
            pltpu.SemaphoreType.DMA,
            pltpu.SemaphoreType.DMA,
            pltpu.SemaphoreType.DMA,
        ],
    )(y, e)


def _prep_body(dp0, dp1, ht, ho, dinv_o, y0_o, hsum_o):
    i = pl.program_id(0)
    deg = jnp.maximum(dp0[...] + dp1[...], 1.0)
    dinv = lax.rsqrt(deg)
    dinv_o[...] = dinv
    y0_o[...] = ht[...] * dinv

    @pl.when(i == 0)
    def _():
        hsum_o[...] = jnp.zeros_like(hsum_o)

    hsum_o[...] += jnp.sum(ho[...], axis=0, keepdims=True)


def _tc_prep(dp0, dp1, h_train, h_ori):
    col1 = pl.BlockSpec((B, 1), lambda i: (i, 0))
    full = pl.BlockSpec((B, D), lambda i: (i, 0))
    return pl.pallas_call(
        _prep_body,
        grid=(GRID,),
        in_specs=[col1, col1, full, full],
        out_specs=[col1, full, pl.BlockSpec((1, D), lambda i: (0, 0))],
        out_shape=[
            jax.ShapeDtypeStruct((N_NODES, 1), jnp.float32),
            jax.ShapeDtypeStruct((N_NODES, D), jnp.float32),
            jax.ShapeDtypeStruct((1, D), jnp.float32),
        ],
    )(dp0, dp1, h_train, h_ori)


def _upd_body_first(x, dinv, p0, p1, l0_o, xn_o, y_o):
    t = dinv[...] * (p0[0] + p1[0])
    xn = x[...] - t
    l0_o[...] = x[...] + t
    xn_o[...] = xn
    y_o[...] = dinv[...] * xn


def _upd_body_mid(x, dinv, p0, p1, xn_o, y_o):
    t = dinv[...] * (p0[0] + p1[0])
    xn = x[...] - t
    xn_o[...] = xn
    y_o[...] = dinv[...] * xn


def _upd_body_last(x, dinv, p0, p1, xn_o):
    xn_o[...] = x[...] - dinv[...] * (p0[0] + p1[0])


def _tc_update(x, dinv, p, kind):
    col1 = pl.BlockSpec((B, 1), lambda i: (i, 0))
    full = pl.BlockSpec((B, D), lambda i: (i, 0))
    part0 = pl.BlockSpec((1, B, D), lambda i: (0, i, 0))
    part1 = pl.BlockSpec((1, B, D), lambda i: (1, i, 0))
    nd = jax.ShapeDtypeStruct((N_NODES, D), jnp.float32)
    body, n_out = {
        "first": (_upd_body_first, 3),
        "mid": (_upd_body_mid, 2),
        "last": (_upd_body_last, 1),
    }[kind]
    return pl.pallas_call(
        body,
        grid=(GRID,),
        in_specs=[full, col1, part0, part1],
        out_specs=[full] * n_out,
        out_shape=[nd] * n_out,
    )(x, dinv, p, p)


def _attn_body(hsum, Wq, bq, Wk, Wv, bv, *refs):
    ls = [r[...] for r in refs[:K_HOPS + 1]]
    out = refs[K_HOPS + 1]
    q = (lax.dot_general(hsum[...], Wq[...], (((1,), (1,)), ((), ())),
                         precision=_HI) * (1.0 / N_NODES)) + bq[...]
    wk = lax.dot_general(q, Wk[...], (((1,), (0,)), ((), ())), precision=_HI)
    s = [jnp.sum(l * wk, axis=1, keepdims=True) * _SCALE for l in ls]
    sc = jnp.concatenate(s, axis=1)
    m = jnp.max(sc, axis=1, keepdims=True)
    e = jnp.exp(sc - m)
    a = e / jnp.sum(e, axis=1, keepdims=True)
    comb = a[:, 0:1] * ls[0]
    for k in range(1, K_HOPS + 1):
        comb = comb + a[:, k:k + 1] * ls[k]
    out[...] = lax.dot_general(comb, Wv[...], (((1,), (1,)), ((), ())),
                               precision=_HI) + bv[...]


def _tc_attn(hsum, Wq, bq, Wk, Wv, bv, ls):
    def fix(shape):
        return pl.BlockSpec(shape, lambda i: tuple(0 for _ in shape))

    full = pl.BlockSpec((B, D), lambda i: (i, 0))
    return pl.pallas_call(
        _attn_body,
        grid=(GRID,),
        in_specs=[fix((1, D)), fix((32, D)), fix((1, 32)), fix((32, D)),
                  fix((D, D)), fix((1, D))] + [full] * (K_HOPS + 1),
        out_specs=full,
        out_shape=jax.ShapeDtypeStruct((N_NODES, D), jnp.float32),
    )(hsum, Wq, bq, Wk, Wv, bv, *ls)


def kernel(edge_index, h_train, h_ori, Wq, bq, Wk, bk, Wv, bv):
    del bk
    row = edge_index[0].astype(jnp.int32).reshape(NW, NCH, C)
    col = edge_index[1].astype(jnp.int32).reshape(NW, NCH, C)
    e = jnp.stack([row, col], axis=2)

    degp = _sc_deg(row)
    dp = degp.reshape(NC, N_PAD, 1)[:, :N_NODES]
    dinv, y, hsum = _tc_prep(dp[0], dp[1], h_train, h_ori)

    x = h_train
    ls = []
    for k in range(K_HOPS):
        p = _sc_hop(y, e)
        if k == 0:
            l0, x, y = _tc_update(x, dinv, p, "first")
            ls += [l0, x]
        elif k < K_HOPS - 1:
            x, y = _tc_update(x, dinv, p, "mid")
            ls.append(x)
        else:
            (x,) = _tc_update(x, dinv, p, "last")
            ls.append(x)

    return _tc_attn(hsum, Wq, bq.reshape(1, -1), Wk, Wv, bv.reshape(1, -1),
                    ls)


# speedup vs baseline: 13.4336x; 1.0104x over previous
import math

import jax
import jax.numpy as jnp
from jax import lax
from jax.experimental import pallas as pl
from jax.experimental.pallas import tpu as pltpu
from jax.experimental.pallas import tpu_sc as plsc

N_NODES = 10000
N_PAD = 10240
D = 128
E = 320000
K_HOPS = 8
NC = 2
NS = 16
NW = NC * NS
E_W = E // NW
C = 80
NCH = E_W // C
N_R = 10240
R_T = N_R // NS
D_T = N_PAD // NS

_SCALE = 1.0 / math.sqrt(D)
_HI = lax.Precision.HIGHEST

B = 2000
GRID = N_NODES // B


def _sc_mesh():
    return plsc.VectorSubcoreMesh(core_axis_name="c", subcore_axis_name="s")


def _deg_body(row_hbm, out_hbm, row_v, ones_v, zb, acc1, sem):
    cid = lax.axis_index("c")
    sid = lax.axis_index("s")
    wid = cid * NS + sid

    for i in range(C // 16):
        ones_v[pl.ds(i * 16, 16)] = jnp.ones((16,), jnp.float32)
    for i in range(D_T // 16):
        zb[pl.ds(i * 16, 16)] = jnp.zeros((16,), jnp.float32)
    pltpu.sync_copy(zb, acc1.at[pl.ds(sid * D_T, D_T)])
    pltpu.sync_copy(row_hbm.at[wid], row_v)
    plsc.subcore_barrier()

    def iss(j, carry):
        pltpu.async_copy(ones_v, acc1.at[row_v.at[j]], sem, add=True)
        return carry

    lax.fori_loop(0, NCH, iss, 0)

    def drain(j, carry):
        pltpu.make_async_copy(ones_v, acc1.at[row_v.at[j]], sem).wait()
        return carry

    lax.fori_loop(0, NCH, drain, 0)
    plsc.subcore_barrier()
    pltpu.sync_copy(acc1.at[pl.ds(sid * D_T, D_T)],
                    out_hbm.at[cid, pl.ds(sid * D_T, D_T)])


def _sc_deg(row):
    return pl.kernel(
        _deg_body,
        mesh=_sc_mesh(),
        out_type=jax.ShapeDtypeStruct((NC, N_PAD), jnp.float32),
        scratch_types=[
            pltpu.VMEM((NCH, C), jnp.int32),
            pltpu.VMEM((C,), jnp.float32),
            pltpu.VMEM((D_T,), jnp.float32),
            pltpu.VMEM_SHARED((N_PAD,), jnp.float32),
            pltpu.SemaphoreType.DMA,
        ],
    )(row)


IB = 5
NB = NCH // IB


def _hop_body(y_hbm, e_hbm, out_hbm, ib, gb0, gb1, gb2, acc, sem_i, sem_g0,
              sem_g1, sem_g2, sem_s0, sem_s1, sem_s2):
    cid = lax.axis_index("c")
    sid = lax.axis_index("s")
    wid = cid * NS + sid

    def zi(i, carry):
        r = i // 8
        c0 = (i - r * 8) * 16
        gb0[r, pl.ds(c0, 16)] = jnp.zeros((16,), jnp.float32)
        return carry

    lax.fori_loop(0, C * 8, zi, 0)

    def zc(k, carry):
        pltpu.sync_copy(gb0, acc.at[pl.ds(sid * R_T + k * C, C)])
        return carry

    lax.fori_loop(0, R_T // C, zc, 0)

    pltpu.sync_copy(e_hbm.at[wid, pl.ds(0, IB)], ib.at[0])
    pltpu.async_copy(e_hbm.at[wid, pl.ds(IB, IB)], ib.at[1], sem_i)
    plsc.subcore_barrier()

    def row_of(c):
        return ib.at[(c // IB) % 3, c % IB, 0]

    def col_of(c):
        return ib.at[(c // IB) % 3, c % IB, 1]

    pltpu.async_copy(y_hbm.at[row_of(0)], gb0, sem_g0)
    pltpu.async_copy(y_hbm.at[row_of(1)], gb1, sem_g1)

    def do_chunk(c, gb, sem_g, sem_s, gb_2, sem_g_2, sem_s_2):
        nx2 = c + 2

        @pl.when(jnp.logical_and(nx2 % IB == 0, nx2 < NCH))
        def _():
            pltpu.make_async_copy(e_hbm.at[0, pl.ds(0, IB)], ib.at[0],
                                  sem_i).wait()
            b = nx2 // IB + 1

            @pl.when(b < NB)
            def _():
                start = jnp.minimum(b * IB, NCH - IB)
                pltpu.async_copy(e_hbm.at[wid, pl.ds(start, IB)],
                                 ib.at[b % 3], sem_i)

        @pl.when(c >= 1)
        def _():
            cp = jnp.maximum(c - 1, 0)
            pltpu.make_async_copy(gb_2, acc.at[col_of(cp)], sem_s_2).wait()

        @pl.when(nx2 < NCH)
        def _():
            pltpu.async_copy(y_hbm.at[row_of(nx2)], gb_2, sem_g_2)

        pltpu.make_async_copy(y_hbm.at[row_of(c)], gb, sem_g).wait()
        pltpu.async_copy(gb, acc.at[col_of(c)], sem_s, add=True)

    def triple(j, carry):
        do_chunk(3 * j, gb0, sem_g0, sem_s0, gb2, sem_g2, sem_s2)
        do_chunk(3 * j + 1, gb1, sem_g1, sem_s1, gb0, sem_g0, sem_s0)
        do_chunk(3 * j + 2, gb2, sem_g2, sem_s2, gb1, sem_g1, sem_s1)
        return carry

    lax.fori_loop(0, NCH // 3, triple, 0)
    do_chunk(NCH - 2, gb0, sem_g0, sem_s0, gb2, sem_g2, sem_s2)
    do_chunk(NCH - 1, gb1, sem_g1, sem_s1, gb0, sem_g0, sem_s0)
    pltpu.make_async_copy(gb1, acc.at[col_of(NCH - 1)], sem_s1).wait()

    plsc.subcore_barrier()
    pltpu.sync_copy(acc.at[pl.ds(sid * R_T, R_T)],
                    out_hbm.at[cid, pl.ds(sid * R_T, R_T)])


def _sc_hop(y, e):
    return pl.kernel(
        _hop_body,
        mesh=_sc_mesh(),
        out_type=jax.ShapeDtypeStruct((NC, N_R, D), jnp.float32),
        scratch_types=[
            pltpu.VMEM((3, IB, 2, C), jnp.int32),
            pltpu.VMEM((C, D), jnp.float32),
            pltpu.VMEM((C, D), jnp.float32),
            pltpu.VMEM((C, D), jnp.float32),
            pltpu.VMEM_SHARED((N_R, D), jnp.float32),
            pltpu.SemaphoreType.DMA,
            pltpu.SemaphoreType.DMA,
            pltpu.SemaphoreType.DMA,
            pltpu.SemaphoreType.DMA,
            pltpu.SemaphoreType.DMA,
            pltpu.SemaphoreType.DMA,
            pltpu.SemaphoreType.DMA,
        ],
    )(y, e)


def _prep_body(dp0, dp1, ht, ho, dinv_o, y0_o, hsum_o):
    i = pl.program_id(0)
    deg = jnp.maximum(dp0[...] + dp1[...], 1.0)
    dinv = lax.rsqrt(deg)
    dinv_o[...] = dinv
    y0_o[...] = ht[...] * dinv

    @pl.when(i == 0)
    def _():
        hsum_o[...] = jnp.zeros_like(hsum_o)

    hsum_o[...] += jnp.sum(ho[...], axis=0, keepdims=True)


def _tc_prep(dp0, dp1, h_train, h_ori):
    col1 = pl.BlockSpec((B, 1), lambda i: (i, 0))
    full = pl.BlockSpec((B, D), lambda i: (i, 0))
    return pl.pallas_call(
        _prep_body,
        grid=(GRID,),
        in_specs=[col1, col1, full, full],
        out_specs=[col1, full, pl.BlockSpec((1, D), lambda i: (0, 0))],
        out_shape=[
            jax.ShapeDtypeStruct((N_NODES, 1), jnp.float32),
            jax.ShapeDtypeStruct((N_NODES, D), jnp.float32),
            jax.ShapeDtypeStruct((1, D), jnp.float32),
        ],
    )(dp0, dp1, h_train, h_ori)


def _upd_body_first(x, dinv, p0, p1, l0_o, xn_o, y_o):
    t = dinv[...] * (p0[0] + p1[0])
    xn = x[...] - t
    l0_o[...] = x[...] + t
    xn_o[...] = xn
    y_o[...] = dinv[...] * xn


def _upd_body_mid(x, dinv, p0, p1, xn_o, y_o):
    t = dinv[...] * (p0[0] + p1[0])
    xn = x[...] - t
    xn_o[...] = xn
    y_o[...] = dinv[...] * xn


def _upd_body_last(x, dinv, p0, p1, xn_o):
    xn_o[...] = x[...] - dinv[...] * (p0[0] + p1[0])


def _tc_update(x, dinv, p, kind):
    col1 = pl.BlockSpec((B, 1), lambda i: (i, 0))
    full = pl.BlockSpec((B, D), lambda i: (i, 0))
    part0 = pl.BlockSpec((1, B, D), lambda i: (0, i, 0))
    part1 = pl.BlockSpec((1, B, D), lambda i: (1, i, 0))
    nd = jax.ShapeDtypeStruct((N_NODES, D), jnp.float32)
    body, n_out = {
        "first": (_upd_body_first, 3),
        "mid": (_upd_body_mid, 2),
        "last": (_upd_body_last, 1),
    }[kind]
    return pl.pallas_call(
        body,
        grid=(GRID,),
        in_specs=[full, col1, part0, part1],
        out_specs=[full] * n_out,
        out_shape=[nd] * n_out,
    )(x, dinv, p, p)


def _attn_body(hsum, Wq, bq, Wk, Wv, bv, x7, dinv, p0, p1, *refs):
    ls = [r[...] for r in refs[:K_HOPS]]
    ls.append(x7[...] - dinv[...] * (p0[0] + p1[0]))
    out = refs[K_HOPS]
    q = (lax.dot_general(hsum[...], Wq[...], (((1,), (1,)), ((), ())),
                         precision=_HI) * (1.0 / N_NODES)) + bq[...]
    wk = lax.dot_general(q, Wk[...], (((1,), (0,)), ((), ())), precision=_HI)
    s = [jnp.sum(l * wk, axis=1, keepdims=True) * _SCALE for l in ls]
    sc = jnp.concatenate(s, axis=1)
    m = jnp.max(sc, axis=1, keepdims=True)
    e = jnp.exp(sc - m)
    a = e / jnp.sum(e, axis=1, keepdims=True)
    comb = a[:, 0:1] * ls[0]
    for k in range(1, K_HOPS + 1):
        comb = comb + a[:, k:k + 1] * ls[k]
    out[...] = lax.dot_general(comb, Wv[...], (((1,), (1,)), ((), ())),
                               precision=_HI) + bv[...]


def _tc_attn(hsum, Wq, bq, Wk, Wv, bv, x7, dinv, p, ls):
    def fix(shape):
        return pl.BlockSpec(shape, lambda i: tuple(0 for _ in shape))

    full = pl.BlockSpec((B, D), lambda i: (i, 0))
    col1 = pl.BlockSpec((B, 1), lambda i: (i, 0))
    part0 = pl.BlockSpec((1, B, D), lambda i: (0, i, 0))
    part1 = pl.BlockSpec((1, B, D), lambda i: (1, i, 0))
    return pl.pallas_call(
        _attn_body,
        grid=(GRID,),
        in_specs=[fix((1, D)), fix((32, D)), fix((1, 32)), fix((32, D)),
                  fix((D, D)), fix((1, D)), full, col1, part0, part1]
                 + [full] * K_HOPS,
        out_specs=full,
        out_shape=jax.ShapeDtypeStruct((N_NODES, D), jnp.float32),
    )(hsum, Wq, bq, Wk, Wv, bv, x7, dinv, p, p, *ls)


def kernel(edge_index, h_train, h_ori, Wq, bq, Wk, bk, Wv, bv):
    del bk
    row = edge_index[0].astype(jnp.int32).reshape(NW, NCH, C)
    col = edge_index[1].astype(jnp.int32).reshape(NW, NCH, C)
    e = jnp.stack([row, col], axis=2)

    degp = _sc_deg(row)
    dp = degp.reshape(NC, N_PAD, 1)[:, :N_NODES]
    dinv, y, hsum = _tc_prep(dp[0], dp[1], h_train, h_ori)

    x = h_train
    ls = []
    for k in range(K_HOPS - 1):
        p = _sc_hop(y, e)
        if k == 0:
            l0, x, y = _tc_update(x, dinv, p, "first")
            ls += [l0, x]
        else:
            x, y = _tc_update(x, dinv, p, "mid")
            ls.append(x)
    p = _sc_hop(y, e)

    return _tc_attn(hsum, Wq, bq.reshape(1, -1), Wk, Wv, bv.reshape(1, -1),
                    x, dinv, p, ls)
